# baseline (device time: 59355 ns/iter reference)
import jax
import jax.numpy as jnp
from jax import lax
from jax.experimental import pallas as pl
from jax.experimental.pallas import tpu as pltpu

N_DEV = 4


def kernel(x, router_W, route_idx, expert_W):
    n_tok, d_model = x.shape
    n_local, _, d_h = expert_W.shape
    n_experts = router_W.shape[1]

    def body(x_ref, rw_ref, idx_ref, ew_ref, out_ref, comm_ref,
             send_sems, recv_sems):
        my = lax.axis_index("i")

        xs = x_ref[:, :]
        scores = jnp.dot(xs, rw_ref[:, :], preferred_element_type=jnp.float32)
        m = jnp.max(scores, axis=1, keepdims=True)
        p = jnp.exp(scores - m)
        p = p / jnp.sum(p, axis=1, keepdims=True)
        e0 = idx_ref[:, 0:1]
        e1 = idx_ref[:, 1:2]
        lane = lax.broadcasted_iota(jnp.int32, p.shape, 1)
        g0 = jnp.sum(jnp.where(lane == e0, p, 0.0), axis=1, keepdims=True)
        g1 = jnp.sum(jnp.where(lane == e1, p, 0.0), axis=1, keepdims=True)
        w0 = g0 / (g0 + g1)
        w1 = g1 / (g0 + g1)

        acc = jnp.zeros((n_tok, d_h), jnp.float32)
        for k in range(n_local):
            e = my * n_local + k
            coef = jnp.where(e0 == e, w0, 0.0) + jnp.where(e1 == e, w1, 0.0)
            xk = (xs * coef).astype(jnp.bfloat16)
            wk = ew_ref[k, :, :].astype(jnp.bfloat16)
            acc = acc + jnp.dot(xk, wk, preferred_element_type=jnp.float32)
        comm_ref[N_DEV - 1, :, :] = acc

        barrier_sem = pltpu.get_barrier_semaphore()
        for dd in range(1, N_DEV):
            peer = (my + dd) % N_DEV
            pl.semaphore_signal(
                barrier_sem, inc=1,
                device_id=(peer,), device_id_type=pl.DeviceIdType.MESH,
            )
        pl.semaphore_wait(barrier_sem, N_DEV - 1)

        rdmas = []
        for dd in range(1, N_DEV):
            peer = (my + dd) % N_DEV
            rdma = pltpu.make_async_remote_copy(
                src_ref=comm_ref.at[N_DEV - 1],
                dst_ref=comm_ref.at[dd - 1],
                send_sem=send_sems.at[dd - 1],
                recv_sem=recv_sems.at[dd - 1],
                device_id=(peer,),
                device_id_type=pl.DeviceIdType.MESH,
            )
            rdma.start()
            rdmas.append(rdma)
        for rdma in rdmas:
            rdma.wait_send()
        for rdma in rdmas:
            rdma.wait_recv()

        out_ref[:, :] = (comm_ref[0, :, :] + comm_ref[1, :, :]
                         + comm_ref[2, :, :] + comm_ref[3, :, :])

    return pl.pallas_call(
        body,
        out_shape=jax.ShapeDtypeStruct((n_tok, d_h), jnp.float32),
        in_specs=[
            pl.BlockSpec(memory_space=pltpu.VMEM),
            pl.BlockSpec(memory_space=pltpu.VMEM),
            pl.BlockSpec(memory_space=pltpu.VMEM),
            pl.BlockSpec(memory_space=pltpu.VMEM),
        ],
        out_specs=pl.BlockSpec(memory_space=pltpu.VMEM),
        scratch_shapes=[
            pltpu.VMEM((N_DEV, n_tok, d_h), jnp.float32),
            pltpu.SemaphoreType.DMA((N_DEV - 1,)),
            pltpu.SemaphoreType.DMA((N_DEV - 1,)),
        ],
        compiler_params=pltpu.CompilerParams(collective_id=0),
    )(x, router_W, route_idx, expert_W)


# device time: 27705 ns/iter; 2.1424x vs baseline; 2.1424x over previous
import jax
import jax.numpy as jnp
from jax import lax
from jax.experimental import pallas as pl
from jax.experimental.pallas import tpu as pltpu

N_DEV = 4


def kernel(x, router_W, route_idx, expert_W):
    n_tok, d_model = x.shape
    n_local, _, d_h = expert_W.shape
    q = n_tok // N_DEV

    def body(x_ref, rw_ref, idx_ref, ew_ref, out_ref,
             acc_ref, rs_send, rs_recv, ag_send, ag_recv,
             rs_send_sems, rs_recv_sems, ag_send_sems, ag_recv_sems):
        my = lax.axis_index("i")

        xs = x_ref[:, :]
        scores = jnp.dot(xs, rw_ref[:, :], preferred_element_type=jnp.float32)
        m = jnp.max(scores, axis=1, keepdims=True)
        p = jnp.exp(scores - m)
        p = p / jnp.sum(p, axis=1, keepdims=True)
        e0 = idx_ref[:, 0:1]
        e1 = idx_ref[:, 1:2]
        lane = lax.broadcasted_iota(jnp.int32, p.shape, 1)
        g0 = jnp.sum(jnp.where(lane == e0, p, 0.0), axis=1, keepdims=True)
        g1 = jnp.sum(jnp.where(lane == e1, p, 0.0), axis=1, keepdims=True)
        w0 = g0 / (g0 + g1)
        w1 = g1 / (g0 + g1)

        acc = jnp.zeros((n_tok, d_h), jnp.float32)
        for k in range(n_local):
            e = my * n_local + k
            coef = jnp.where(e0 == e, w0, 0.0) + jnp.where(e1 == e, w1, 0.0)
            xk = (xs * coef).astype(jnp.bfloat16)
            wk = ew_ref[k, :, :].astype(jnp.bfloat16)
            acc = acc + jnp.dot(xk, wk, preferred_element_type=jnp.float32)

        acc_ref[:, :] = acc
        for dd in range(1, N_DEV):
            peer = (my + dd) % N_DEV
            rs_send[dd - 1, :, :] = acc_ref[pl.ds(peer * q, q), :].astype(
                jnp.bfloat16)

        barrier_sem = pltpu.get_barrier_semaphore()
        for dd in range(1, N_DEV):
            peer = (my + dd) % N_DEV
            pl.semaphore_signal(
                barrier_sem, inc=1,
                device_id=(peer,), device_id_type=pl.DeviceIdType.MESH,
            )
        pl.semaphore_wait(barrier_sem, N_DEV - 1)

        rs_rdmas = []
        for dd in range(1, N_DEV):
            peer = (my + dd) % N_DEV
            rdma = pltpu.make_async_remote_copy(
                src_ref=rs_send.at[dd - 1],
                dst_ref=rs_recv.at[dd - 1],
                send_sem=rs_send_sems.at[dd - 1],
                recv_sem=rs_recv_sems.at[dd - 1],
                device_id=(peer,),
                device_id_type=pl.DeviceIdType.MESH,
            )
            rdma.start()
            rs_rdmas.append(rdma)
        for rdma in rs_rdmas:
            rdma.wait_recv()

        own = acc_ref[pl.ds(my * q, q), :]
        red = (own
               + rs_recv[0, :, :].astype(jnp.float32)
               + rs_recv[1, :, :].astype(jnp.float32)
               + rs_recv[2, :, :].astype(jnp.float32))
        out_ref[pl.ds(my * q, q), :] = red
        ag_send[:, :] = red.astype(jnp.bfloat16)

        ag_rdmas = []
        for dd in range(1, N_DEV):
            peer = (my + dd) % N_DEV
            rdma = pltpu.make_async_remote_copy(
                src_ref=ag_send,
                dst_ref=ag_recv.at[dd - 1],
                send_sem=ag_send_sems.at[dd - 1],
                recv_sem=ag_recv_sems.at[dd - 1],
                device_id=(peer,),
                device_id_type=pl.DeviceIdType.MESH,
            )
            rdma.start()
            ag_rdmas.append(rdma)
        for dd in range(1, N_DEV):
            ag_rdmas[dd - 1].wait_recv()
            src = (my - dd) % N_DEV
            out_ref[pl.ds(src * q, q), :] = ag_recv[dd - 1, :, :].astype(
                jnp.float32)

        for rdma in rs_rdmas:
            rdma.wait_send()
        for rdma in ag_rdmas:
            rdma.wait_send()

    return pl.pallas_call(
        body,
        out_shape=jax.ShapeDtypeStruct((n_tok, d_h), jnp.float32),
        in_specs=[
            pl.BlockSpec(memory_space=pltpu.VMEM),
            pl.BlockSpec(memory_space=pltpu.VMEM),
            pl.BlockSpec(memory_space=pltpu.VMEM),
            pl.BlockSpec(memory_space=pltpu.VMEM),
        ],
        out_specs=pl.BlockSpec(memory_space=pltpu.VMEM),
        scratch_shapes=[
            pltpu.VMEM((n_tok, d_h), jnp.float32),
            pltpu.VMEM((N_DEV - 1, q, d_h), jnp.bfloat16),
            pltpu.VMEM((N_DEV - 1, q, d_h), jnp.bfloat16),
            pltpu.VMEM((q, d_h), jnp.bfloat16),
            pltpu.VMEM((N_DEV - 1, q, d_h), jnp.bfloat16),
            pltpu.SemaphoreType.DMA((N_DEV - 1,)),
            pltpu.SemaphoreType.DMA((N_DEV - 1,)),
            pltpu.SemaphoreType.DMA((N_DEV - 1,)),
            pltpu.SemaphoreType.DMA((N_DEV - 1,)),
        ],
        compiler_params=pltpu.CompilerParams(collective_id=0),
    )(x, router_W, route_idx, expert_W)


# device time: 27366 ns/iter; 2.1689x vs baseline; 1.0124x over previous
import jax
import jax.numpy as jnp
from jax import lax
from jax.experimental import pallas as pl
from jax.experimental.pallas import tpu as pltpu

N_DEV = 4


def kernel(x, router_W, route_idx, expert_W):
    n_tok, d_model = x.shape
    n_local, _, d_h = expert_W.shape
    q = n_tok // N_DEV

    def body(x_ref, rw_ref, idx_ref, ew_ref, out_ref,
             coef_ref, rs_send, rs_recv, ag_send, ag_recv,
             rs_send_sems, rs_recv_sems, ag_send_sems, ag_recv_sems):
        my = lax.axis_index("i")

        barrier_sem = pltpu.get_barrier_semaphore()
        for dd in range(1, N_DEV):
            peer = (my + dd) % N_DEV
            pl.semaphore_signal(
                barrier_sem, inc=1,
                device_id=(peer,), device_id_type=pl.DeviceIdType.MESH,
            )
        pl.semaphore_wait(barrier_sem, N_DEV - 1)

        xs = x_ref[:, :]
        scores = jnp.dot(xs, rw_ref[:, :], preferred_element_type=jnp.float32)
        m = jnp.max(scores, axis=1, keepdims=True)
        p = jnp.exp(scores - m)
        p = p / jnp.sum(p, axis=1, keepdims=True)
        e0 = idx_ref[:, 0:1]
        e1 = idx_ref[:, 1:2]
        lane = lax.broadcasted_iota(jnp.int32, p.shape, 1)
        g0 = jnp.sum(jnp.where(lane == e0, p, 0.0), axis=1, keepdims=True)
        g1 = jnp.sum(jnp.where(lane == e1, p, 0.0), axis=1, keepdims=True)
        w0 = g0 / (g0 + g1)
        w1 = g1 / (g0 + g1)
        for k in range(n_local):
            e = my * n_local + k
            coef_ref[:, k:k + 1] = (jnp.where(e0 == e, w0, 0.0)
                                    + jnp.where(e1 == e, w1, 0.0))

        def quarter_partial(row0):
            xq = x_ref[pl.ds(row0, q), :]
            accq = jnp.zeros((q, d_h), jnp.float32)
            for k in range(n_local):
                ck = coef_ref[pl.ds(row0, q), k:k + 1]
                xk = (xq * ck).astype(jnp.bfloat16)
                accq = accq + jnp.dot(xk, ew_ref[k, :, :].astype(jnp.bfloat16),
                                      preferred_element_type=jnp.float32)
            return accq

        rs_rdmas = {}
        for dd in (2, 1, 3):
            peer = (my + dd) % N_DEV
            rs_send[dd - 1, :, :] = quarter_partial(peer * q).astype(
                jnp.bfloat16)
            rdma = pltpu.make_async_remote_copy(
                src_ref=rs_send.at[dd - 1],
                dst_ref=rs_recv.at[dd - 1],
                send_sem=rs_send_sems.at[dd - 1],
                recv_sem=rs_recv_sems.at[dd - 1],
                device_id=(peer,),
                device_id_type=pl.DeviceIdType.MESH,
            )
            rdma.start()
            rs_rdmas[dd] = rdma

        own = quarter_partial(my * q)

        for dd in (1, 2, 3):
            rs_rdmas[dd].wait_recv()
        red = (own
               + rs_recv[0, :, :].astype(jnp.float32)
               + rs_recv[1, :, :].astype(jnp.float32)
               + rs_recv[2, :, :].astype(jnp.float32))
        out_ref[pl.ds(my * q, q), :] = red
        ag_send[:, :] = red.astype(jnp.bfloat16)

        ag_rdmas = {}
        for dd in (2, 1, 3):
            peer = (my + dd) % N_DEV
            rdma = pltpu.make_async_remote_copy(
                src_ref=ag_send,
                dst_ref=ag_recv.at[dd - 1],
                send_sem=ag_send_sems.at[dd - 1],
                recv_sem=ag_recv_sems.at[dd - 1],
                device_id=(peer,),
                device_id_type=pl.DeviceIdType.MESH,
            )
            rdma.start()
            ag_rdmas[dd] = rdma
        for dd in (1, 2, 3):
            ag_rdmas[dd].wait_recv()
            src = (my - dd) % N_DEV
            out_ref[pl.ds(src * q, q), :] = ag_recv[dd - 1, :, :].astype(
                jnp.float32)

        for dd in (1, 2, 3):
            rs_rdmas[dd].wait_send()
            ag_rdmas[dd].wait_send()

    return pl.pallas_call(
        body,
        out_shape=jax.ShapeDtypeStruct((n_tok, d_h), jnp.float32),
        in_specs=[
            pl.BlockSpec(memory_space=pltpu.VMEM),
            pl.BlockSpec(memory_space=pltpu.VMEM),
            pl.BlockSpec(memory_space=pltpu.VMEM),
            pl.BlockSpec(memory_space=pltpu.VMEM),
        ],
        out_specs=pl.BlockSpec(memory_space=pltpu.VMEM),
        scratch_shapes=[
            pltpu.VMEM((n_tok, n_local), jnp.float32),
            pltpu.VMEM((N_DEV - 1, q, d_h), jnp.bfloat16),
            pltpu.VMEM((N_DEV - 1, q, d_h), jnp.bfloat16),
            pltpu.VMEM((q, d_h), jnp.bfloat16),
            pltpu.VMEM((N_DEV - 1, q, d_h), jnp.bfloat16),
            pltpu.SemaphoreType.DMA((N_DEV - 1,)),
            pltpu.SemaphoreType.DMA((N_DEV - 1,)),
            pltpu.SemaphoreType.DMA((N_DEV - 1,)),
            pltpu.SemaphoreType.DMA((N_DEV - 1,)),
        ],
        compiler_params=pltpu.CompilerParams(collective_id=0),
    )(x, router_W, route_idx, expert_W)


# device time: 20613 ns/iter; 2.8795x vs baseline; 1.3276x over previous
import jax
import jax.numpy as jnp
from jax import lax
from jax.experimental import pallas as pl
from jax.experimental.pallas import tpu as pltpu

N_DEV = 4
N_HALF = 2


def kernel(x, router_W, route_idx, expert_W):
    n_tok, d_model = x.shape
    n_local, _, d_h = expert_W.shape
    q = n_tok // N_DEV
    hw = d_h // N_HALF

    def body(x_hbm, rwt_hbm, idxt_hbm, ew_hbm, out_hbm,
             x_ref, ew_ref, rwt_ref, idxt_ref, own_ref, stage_ref,
             rs_send, rs_recv, ag_send, ag_recv,
             in_sems, out_sems,
             rs_send_sems, rs_recv_sems, ag_send_sems, ag_recv_sems):
        my = lax.axis_index("i")

        quarter_order = [(my + dd) % N_DEV for dd in (2, 1, 3)] + [my]
        x_dmas = {}
        for i, qidx in enumerate(quarter_order):
            rows = pl.ds(qidx * q, q)
            x_dmas[i] = pltpu.make_async_copy(
                x_hbm.at[rows, :], x_ref.at[rows, :], in_sems.at[4 + i])
        ew_dma = pltpu.make_async_copy(ew_hbm, ew_ref, in_sems.at[1])
        rwt_dma = pltpu.make_async_copy(rwt_hbm, rwt_ref, in_sems.at[2])
        idxt_dma = pltpu.make_async_copy(idxt_hbm, idxt_ref, in_sems.at[3])
        rwt_dma.start()
        idxt_dma.start()
        for i in range(N_DEV):
            x_dmas[i].start()
        ew_dma.start()

        barrier_sem = pltpu.get_barrier_semaphore()
        for dd in range(1, N_DEV):
            peer = (my + dd) % N_DEV
            pl.semaphore_signal(
                barrier_sem, inc=1,
                device_id=(peer,), device_id_type=pl.DeviceIdType.MESH,
            )

        rwt_dma.wait()
        idxt_dma.wait()

        def quarter_partial(row0):
            xq = x_ref[pl.ds(row0, q), :]
            scores = lax.dot_general(
                xq, rwt_ref[:, :], (((1,), (1,)), ((), ())),
                preferred_element_type=jnp.float32)
            m = jnp.max(scores, axis=1, keepdims=True)
            p = jnp.exp(scores - m)
            p = p / jnp.sum(p, axis=1, keepdims=True)
            e0 = jnp.transpose(idxt_ref[0:1, pl.ds(row0, q)])
            e1 = jnp.transpose(idxt_ref[1:2, pl.ds(row0, q)])
            lane = lax.broadcasted_iota(jnp.int32, p.shape, 1)
            g0 = jnp.sum(jnp.where(lane == e0, p, 0.0), axis=1, keepdims=True)
            g1 = jnp.sum(jnp.where(lane == e1, p, 0.0), axis=1, keepdims=True)
            w0 = g0 / (g0 + g1)
            w1 = g1 / (g0 + g1)
            accq = jnp.zeros((q, d_h), jnp.float32)
            for k in range(n_local):
                ck = (jnp.where(e0 == my * n_local + k, w0, 0.0)
                      + jnp.where(e1 == my * n_local + k, w1, 0.0))
                xk = (xq * ck).astype(jnp.bfloat16)
                accq = accq + jnp.dot(xk, ew_ref[k, :, :].astype(jnp.bfloat16),
                                      preferred_element_type=jnp.float32)
            return accq

        def remote_copy(src, dst, ssem, rsem, peer):
            return pltpu.make_async_remote_copy(
                src_ref=src, dst_ref=dst, send_sem=ssem, recv_sem=rsem,
                device_id=(peer,), device_id_type=pl.DeviceIdType.MESH,
            )

        rs_rdmas = {}
        for i, dd in enumerate((2, 1, 3)):
            peer = (my + dd) % N_DEV
            x_dmas[i].wait()
            if dd == 2:
                ew_dma.wait()
            rs_send[dd - 1, :, :] = quarter_partial(peer * q).astype(
                jnp.bfloat16)
            if dd == 2:
                pl.semaphore_wait(barrier_sem, N_DEV - 1)
            for h in range(N_HALF):
                cols = pl.ds(h * hw, hw)
                rdma = remote_copy(
                    rs_send.at[dd - 1, :, cols], rs_recv.at[dd - 1, :, cols],
                    rs_send_sems.at[dd - 1, h], rs_recv_sems.at[dd - 1, h],
                    peer)
                rdma.start()
                rs_rdmas[(dd, h)] = rdma

        x_dmas[3].wait()
        own_ref[:, :] = quarter_partial(my * q)

        out_dmas = []
        ag_rdmas = {}
        for h in range(N_HALF):
            cols = pl.ds(h * hw, hw)
            for dd in (1, 3, 2):
                rs_rdmas[(dd, h)].wait_recv()
            red = (own_ref[:, cols]
                   + rs_recv[0, :, cols].astype(jnp.float32)
                   + rs_recv[1, :, cols].astype(jnp.float32)
                   + rs_recv[2, :, cols].astype(jnp.float32))
            ag_send[:, cols] = red.astype(jnp.bfloat16)
            for dd in (2, 1, 3):
                peer = (my + dd) % N_DEV
                rdma = remote_copy(
                    ag_send.at[:, cols], ag_recv.at[dd - 1, :, cols],
                    ag_send_sems.at[dd - 1, h], ag_recv_sems.at[dd - 1, h],
                    peer)
                rdma.start()
                ag_rdmas[(dd, h)] = rdma
            rows = pl.ds(my * q, q)
            stage_ref[rows, cols] = red
            dma = pltpu.make_async_copy(
                stage_ref.at[rows, cols], out_hbm.at[rows, cols],
                out_sems.at[N_DEV - 1, h])
            dma.start()
            out_dmas.append(dma)

        for h in range(N_HALF):
            cols = pl.ds(h * hw, hw)
            for dd in (1, 3, 2):
                ag_rdmas[(dd, h)].wait_recv()
                src = (my - dd) % N_DEV
                rows = pl.ds(src * q, q)
                stage_ref[rows, cols] = ag_recv[dd - 1, :, cols].astype(
                    jnp.float32)
                dma = pltpu.make_async_copy(
                    stage_ref.at[rows, cols], out_hbm.at[rows, cols],
                    out_sems.at[dd - 1, h])
                dma.start()
                out_dmas.append(dma)

        for dma in out_dmas:
            dma.wait()
        for key, rdma in rs_rdmas.items():
            rdma.wait_send()
        for key, rdma in ag_rdmas.items():
            rdma.wait_send()

    grid_spec = pltpu.PrefetchScalarGridSpec(
        num_scalar_prefetch=0,
        in_specs=[
            pl.BlockSpec(memory_space=pltpu.HBM),
            pl.BlockSpec(memory_space=pltpu.HBM),
            pl.BlockSpec(memory_space=pltpu.HBM),
            pl.BlockSpec(memory_space=pltpu.HBM),
        ],
        out_specs=pl.BlockSpec(memory_space=pltpu.HBM),
        scratch_shapes=[
            pltpu.VMEM((n_tok, d_model), jnp.float32),
            pltpu.VMEM((n_local, d_model, d_h), jnp.float32),
            pltpu.VMEM((16, d_model), jnp.float32),
            pltpu.VMEM((2, n_tok), jnp.int32),
            pltpu.VMEM((q, d_h), jnp.float32),
            pltpu.VMEM((n_tok, d_h), jnp.float32),
            pltpu.VMEM((N_DEV - 1, q, d_h), jnp.bfloat16),
            pltpu.VMEM((N_DEV - 1, q, d_h), jnp.bfloat16),
            pltpu.VMEM((q, d_h), jnp.bfloat16),
            pltpu.VMEM((N_DEV - 1, q, d_h), jnp.bfloat16),
            pltpu.SemaphoreType.DMA((8,)),
            pltpu.SemaphoreType.DMA((N_DEV, N_HALF)),
            pltpu.SemaphoreType.DMA((N_DEV - 1, N_HALF)),
            pltpu.SemaphoreType.DMA((N_DEV - 1, N_HALF)),
            pltpu.SemaphoreType.DMA((N_DEV - 1, N_HALF)),
            pltpu.SemaphoreType.DMA((N_DEV - 1, N_HALF)),
        ],
    )

    hbm = lambda a: pltpu.with_memory_space_constraint(a, pltpu.MemorySpace.HBM)
    return pl.pallas_call(
        body,
        out_shape=jax.ShapeDtypeStruct((n_tok, d_h), jnp.float32),
        grid_spec=grid_spec,
        compiler_params=pltpu.CompilerParams(collective_id=0),
    )(hbm(x), hbm(router_W.T), hbm(route_idx.T), hbm(expert_W))
